# baseline (speedup 1.0000x reference)
import functools

import jax
import jax.numpy as jnp
from jax import lax
from jax.experimental import pallas as pl
from jax.experimental.pallas import tpu as pltpu
from jax.experimental.pallas import tpu_sc as plsc

VOCAB = 1000000
BATCH = 4096
SEQ = 200
D = 64
DP = 128

SPLIT = 500224
TBLK = 512
TGRID = SPLIT // TBLK
assert TGRID * TBLK == SPLIT

NC, NS = 2, 16
NW = NC * NS
ROWS_W = BATCH // NW
GA = 128
GB = SEQ - GA
NBUF = 4


def _tc_linearize(table_t):
    def body(a_ref, b_ref, o_ref):
        o_ref[:, 0:D] = a_ref[...].T
        o_ref[:, D:DP] = b_ref[...].T

    return pl.pallas_call(
        body,
        grid=(TGRID,),
        in_specs=[
            pl.BlockSpec((D, TBLK), lambda j: (0, j)),
            pl.BlockSpec((D, TBLK), lambda j: (0, j + TGRID)),
        ],
        out_specs=pl.BlockSpec((TBLK, DP), lambda j: (j, 0)),
        out_shape=jax.ShapeDtypeStruct((SPLIT, DP), jnp.float32),
    )(table_t, table_t)


def _make_sc_gather():
    mesh = plsc.VectorSubcoreMesh(core_axis_name="c", subcore_axis_name="s")

    @functools.partial(
        pl.kernel,
        mesh=mesh,
        compiler_params=pltpu.CompilerParams(use_tc_tiling_on_sc=False),
        out_type=jax.ShapeDtypeStruct((BATCH, SEQ, DP), jnp.float32),
        scratch_types=(
            [pltpu.VMEM((ROWS_W, GA), jnp.int32), pltpu.VMEM((ROWS_W, GB), jnp.int32)]
            + [pltpu.VMEM((SEQ, D), jnp.float32) for _ in range(NBUF)]
            + [pltpu.SemaphoreType.DMA for _ in range(2 * NBUF)]
        ),
    )
    def k(idxa_hbm, idxb_hbm, table_hbm, out_hbm, idxa_v, idxb_v, *bufs_and_sems):
        bufs = bufs_and_sems[:NBUF]
        gsem = bufs_and_sems[NBUF : 2 * NBUF]
        wsem = bufs_and_sems[2 * NBUF : 3 * NBUF]

        wid = lax.axis_index("s") * NC + lax.axis_index("c")
        row0 = wid * ROWS_W
        pltpu.sync_copy(idxa_hbm.at[pl.ds(row0, ROWS_W)], idxa_v)
        pltpu.sync_copy(idxb_hbm.at[pl.ds(row0, ROWS_W), pl.ds(0, GB)], idxb_v)

        def out_dst(r):
            return out_hbm.at[row0 + r, :, pl.ds(0, D)]

        def issue_gathers(r, p):
            pltpu.async_copy(
                table_hbm.at[idxa_v.at[r]],
                bufs[p].at[pl.ds(0, GA)],
                gsem[p],
            )
            pltpu.async_copy(
                table_hbm.at[idxb_v.at[r]],
                bufs[p].at[pl.ds(GA, GB)],
                gsem[p],
            )

        def wait_gathers(p):
            pltpu.make_async_copy(
                table_hbm.at[idxa_v.at[0]],
                bufs[p].at[pl.ds(0, GA)],
                gsem[p],
            ).wait()
            pltpu.make_async_copy(
                table_hbm.at[idxb_v.at[0]],
                bufs[p].at[pl.ds(GA, GB)],
                gsem[p],
            ).wait()

        def wait_write(p):
            pltpu.make_async_copy(bufs[p], out_dst(0), wsem[p]).wait()

        issue_gathers(0, 0)
        issue_gathers(1, 1)

        def body(m, carry):
            for j in range(NBUF):
                r = NBUF * m + j
                p = j
                p2 = (j + 2) % NBUF
                wait_gathers(p)
                pltpu.async_copy(bufs[p], out_dst(r), wsem[p])

                @pl.when(r >= 2)
                def _():
                    wait_write(p2)

                @pl.when(r + 2 < ROWS_W)
                def _():
                    issue_gathers(r + 2, p2)

            return carry

        lax.fori_loop(0, ROWS_W // NBUF, body, 0)
        wait_write(2)
        wait_write(3)

    return k


_sc_gather = _make_sc_gather()


def kernel(word_indices, word_embedding_weight):
    idx = word_indices.astype(jnp.int32)
    idx = jnp.where(idx < SPLIT, idx * 2, (idx - SPLIT) * 2 + 1)
    idxa = idx[:, :GA]
    idxb = jnp.pad(idx[:, GA:], ((0, 0), (0, GA - GB)))
    packed = _tc_linearize(word_embedding_weight.T)
    table_lin = packed.reshape(2 * SPLIT, D)
    out = _sc_gather(idxa, idxb, table_lin)
    return out[:, :, :D]


# speedup vs baseline: 1.3926x; 1.3926x over previous
import functools

import jax
import jax.numpy as jnp
from jax import lax
from jax.experimental import pallas as pl
from jax.experimental.pallas import tpu as pltpu
from jax.experimental.pallas import tpu_sc as plsc

VOCAB = 1000000
BATCH = 4096
SEQ = 200
D = 64
DP = 128

SPLIT = 500480
TBLK = 1280
TGRID = SPLIT // TBLK
assert TGRID * TBLK == SPLIT

NC, NS = 2, 16
NW = NC * NS
ROWS_W = BATCH // NW
GA = 128
GB = SEQ - GA
NBUF = 4


def _tc_linearize(table_t):
    def body(a_ref, b_ref, o_ref):
        o_ref[:, 0:D] = a_ref[...].T
        o_ref[:, D:DP] = b_ref[...].T

    return pl.pallas_call(
        body,
        grid=(TGRID,),
        in_specs=[
            pl.BlockSpec((D, TBLK), lambda j: (0, j)),
            pl.BlockSpec((D, TBLK), lambda j: (0, j + TGRID)),
        ],
        out_specs=pl.BlockSpec((TBLK, DP), lambda j: (j, 0)),
        out_shape=jax.ShapeDtypeStruct((SPLIT, DP), jnp.float32),
    )(table_t, table_t)


def _make_sc_gather():
    mesh = plsc.VectorSubcoreMesh(core_axis_name="c", subcore_axis_name="s")

    @functools.partial(
        pl.kernel,
        mesh=mesh,
        compiler_params=pltpu.CompilerParams(use_tc_tiling_on_sc=False),
        out_type=jax.ShapeDtypeStruct((BATCH, SEQ, DP), jnp.float32),
        scratch_types=(
            [pltpu.VMEM((ROWS_W, GA), jnp.int32), pltpu.VMEM((ROWS_W, GB), jnp.int32)]
            + [pltpu.VMEM((SEQ, D), jnp.float32) for _ in range(NBUF)]
            + [pltpu.SemaphoreType.DMA for _ in range(2 * NBUF)]
        ),
    )
    def k(idxa_hbm, idxb_hbm, table_hbm, out_hbm, idxa_v, idxb_v, *bufs_and_sems):
        bufs = bufs_and_sems[:NBUF]
        gsem = bufs_and_sems[NBUF : 2 * NBUF]
        wsem = bufs_and_sems[2 * NBUF : 3 * NBUF]

        wid = lax.axis_index("s") * NC + lax.axis_index("c")
        row0 = wid * ROWS_W
        pltpu.sync_copy(idxa_hbm.at[pl.ds(row0, ROWS_W)], idxa_v)
        pltpu.sync_copy(idxb_hbm.at[pl.ds(row0, ROWS_W), pl.ds(0, GB)], idxb_v)

        def out_dst(r):
            return out_hbm.at[row0 + r, :, pl.ds(0, D)]

        def issue_gathers(r, p):
            pltpu.async_copy(
                table_hbm.at[idxa_v.at[r]],
                bufs[p].at[pl.ds(0, GA)],
                gsem[p],
            )
            pltpu.async_copy(
                table_hbm.at[idxb_v.at[r]],
                bufs[p].at[pl.ds(GA, GB)],
                gsem[p],
            )

        def wait_gathers(p):
            pltpu.make_async_copy(
                table_hbm.at[idxa_v.at[0]],
                bufs[p].at[pl.ds(0, GA)],
                gsem[p],
            ).wait()
            pltpu.make_async_copy(
                table_hbm.at[idxb_v.at[0]],
                bufs[p].at[pl.ds(GA, GB)],
                gsem[p],
            ).wait()

        def wait_write(p):
            pltpu.make_async_copy(bufs[p], out_dst(0), wsem[p]).wait()

        issue_gathers(0, 0)
        issue_gathers(1, 1)

        def body(m, carry):
            for j in range(NBUF):
                r = NBUF * m + j
                p = j
                p2 = (j + 2) % NBUF
                wait_gathers(p)
                pltpu.async_copy(bufs[p], out_dst(r), wsem[p])

                @pl.when(r >= 2)
                def _():
                    wait_write(p2)

                @pl.when(r + 2 < ROWS_W)
                def _():
                    issue_gathers(r + 2, p2)

            return carry

        lax.fori_loop(0, ROWS_W // NBUF, body, 0)
        wait_write(2)
        wait_write(3)

    return k


_sc_gather = _make_sc_gather()


def kernel(word_indices, word_embedding_weight):
    idx = word_indices.astype(jnp.int32)
    idx = jnp.where(idx < SPLIT, idx * 2, (idx - SPLIT) * 2 + 1)
    idxa = idx[:, :GA]
    idxb = jnp.pad(idx[:, GA:], ((0, 0), (0, GA - GB)))
    packed = _tc_linearize(word_embedding_weight.T)
    table_lin = packed.reshape(2 * SPLIT, D)
    out = _sc_gather(idxa, idxb, table_lin)
    return out[:, :, :D]


# speedup vs baseline: 1.7633x; 1.2662x over previous
import functools

import jax
import jax.numpy as jnp
from jax import lax
from jax.experimental import pallas as pl
from jax.experimental.pallas import tpu as pltpu
from jax.experimental.pallas import tpu_sc as plsc

VOCAB = 1000000
BATCH = 4096
SEQ = 200
D = 64
DP = 128

SPLIT = 501760
TBLK = 5120
TGRID = SPLIT // TBLK
assert TGRID * TBLK == SPLIT

NC, NS = 2, 16
NW = NC * NS
ROWS_W = BATCH // NW
GA = 128
GB = SEQ - GA
NBUF = 4


def _tc_linearize(table_t):
    def body(a_ref, b_ref, o_ref):
        o_ref[:, 0:D] = a_ref[...].T
        o_ref[:, D:DP] = b_ref[...].T

    return pl.pallas_call(
        body,
        grid=(TGRID,),
        in_specs=[
            pl.BlockSpec((D, TBLK), lambda j: (0, j)),
            pl.BlockSpec((D, TBLK), lambda j: (0, j + TGRID)),
        ],
        out_specs=pl.BlockSpec((TBLK, DP), lambda j: (j, 0)),
        out_shape=jax.ShapeDtypeStruct((SPLIT, DP), jnp.float32),
    )(table_t, table_t)


def _make_sc_gather():
    mesh = plsc.VectorSubcoreMesh(core_axis_name="c", subcore_axis_name="s")

    @functools.partial(
        pl.kernel,
        mesh=mesh,
        compiler_params=pltpu.CompilerParams(use_tc_tiling_on_sc=False),
        out_type=jax.ShapeDtypeStruct((BATCH, SEQ, DP), jnp.float32),
        scratch_types=(
            [pltpu.VMEM((ROWS_W, GA), jnp.int32), pltpu.VMEM((ROWS_W, GB), jnp.int32)]
            + [pltpu.VMEM((SEQ, D), jnp.float32) for _ in range(NBUF)]
            + [pltpu.SemaphoreType.DMA for _ in range(2 * NBUF)]
        ),
    )
    def k(idxa_hbm, idxb_hbm, table_hbm, out_hbm, idxa_v, idxb_v, *bufs_and_sems):
        bufs = bufs_and_sems[:NBUF]
        gsem = bufs_and_sems[NBUF : 2 * NBUF]
        wsem = bufs_and_sems[2 * NBUF : 3 * NBUF]

        wid = lax.axis_index("s") * NC + lax.axis_index("c")
        row0 = wid * ROWS_W
        pltpu.sync_copy(idxa_hbm.at[pl.ds(row0, ROWS_W)], idxa_v)
        pltpu.sync_copy(idxb_hbm.at[pl.ds(row0, ROWS_W), pl.ds(0, GB)], idxb_v)

        def out_dst(r):
            return out_hbm.at[row0 + r, :, pl.ds(0, D)]

        def issue_gathers(r, p):
            pltpu.async_copy(
                table_hbm.at[idxa_v.at[r]],
                bufs[p].at[pl.ds(0, GA)],
                gsem[p],
            )
            pltpu.async_copy(
                table_hbm.at[idxb_v.at[r]],
                bufs[p].at[pl.ds(GA, GB)],
                gsem[p],
            )

        def wait_gathers(p):
            pltpu.make_async_copy(
                table_hbm.at[idxa_v.at[0]],
                bufs[p].at[pl.ds(0, GA)],
                gsem[p],
            ).wait()
            pltpu.make_async_copy(
                table_hbm.at[idxb_v.at[0]],
                bufs[p].at[pl.ds(GA, GB)],
                gsem[p],
            ).wait()

        def wait_write(p):
            pltpu.make_async_copy(bufs[p], out_dst(0), wsem[p]).wait()

        issue_gathers(0, 0)
        issue_gathers(1, 1)

        def body(m, carry):
            for j in range(NBUF):
                r = NBUF * m + j
                p = j
                p2 = (j + 2) % NBUF
                wait_gathers(p)
                pltpu.async_copy(bufs[p], out_dst(r), wsem[p])

                @pl.when(r >= 2)
                def _():
                    wait_write(p2)

                @pl.when(r + 2 < ROWS_W)
                def _():
                    issue_gathers(r + 2, p2)

            return carry

        lax.fori_loop(0, ROWS_W // NBUF, body, 0)
        wait_write(2)
        wait_write(3)

    return k


_sc_gather = _make_sc_gather()


def kernel(word_indices, word_embedding_weight):
    idx = word_indices.astype(jnp.int32)
    idx = jnp.where(idx < SPLIT, idx * 2, (idx - SPLIT) * 2 + 1)
    idxa = idx[:, :GA]
    idxb = jnp.pad(idx[:, GA:], ((0, 0), (0, GA - GB)))
    packed = _tc_linearize(word_embedding_weight.T)
    table_lin = packed.reshape(2 * SPLIT, D)
    out = _sc_gather(idxa, idxb, table_lin)
    return out[:, :, :D]


# speedup vs baseline: 1.8482x; 1.0482x over previous
import functools

import jax
import jax.numpy as jnp
from jax import lax
from jax.experimental import pallas as pl
from jax.experimental.pallas import tpu as pltpu
from jax.experimental.pallas import tpu_sc as plsc

VOCAB = 1000000
BATCH = 4096
SEQ = 200
D = 64
DP = 128

SPLIT = 501760
TBLK = 10240
TGRID = SPLIT // TBLK
assert TGRID * TBLK == SPLIT

NC, NS = 2, 16
NW = NC * NS
ROWS_W = BATCH // NW
GA = 128
GB = SEQ - GA
NBUF = 4


def _tc_linearize(table_t):
    def body(a_ref, b_ref, o_ref):
        o_ref[:, 0:D] = a_ref[...].T
        o_ref[:, D:DP] = b_ref[...].T

    return pl.pallas_call(
        body,
        grid=(TGRID,),
        in_specs=[
            pl.BlockSpec((D, TBLK), lambda j: (0, j)),
            pl.BlockSpec((D, TBLK), lambda j: (0, j + TGRID)),
        ],
        out_specs=pl.BlockSpec((TBLK, DP), lambda j: (j, 0)),
        out_shape=jax.ShapeDtypeStruct((SPLIT, DP), jnp.float32),
    )(table_t, table_t)


def _make_sc_gather():
    mesh = plsc.VectorSubcoreMesh(core_axis_name="c", subcore_axis_name="s")

    @functools.partial(
        pl.kernel,
        mesh=mesh,
        compiler_params=pltpu.CompilerParams(use_tc_tiling_on_sc=False),
        out_type=jax.ShapeDtypeStruct((BATCH, SEQ, DP), jnp.float32),
        scratch_types=(
            [pltpu.VMEM((ROWS_W, GA), jnp.int32), pltpu.VMEM((ROWS_W, GB), jnp.int32)]
            + [pltpu.VMEM((SEQ, D), jnp.float32) for _ in range(NBUF)]
            + [pltpu.SemaphoreType.DMA for _ in range(2 * NBUF)]
        ),
    )
    def k(idxa_hbm, idxb_hbm, table_hbm, out_hbm, idxa_v, idxb_v, *bufs_and_sems):
        bufs = bufs_and_sems[:NBUF]
        gsem = bufs_and_sems[NBUF : 2 * NBUF]
        wsem = bufs_and_sems[2 * NBUF : 3 * NBUF]

        wid = lax.axis_index("s") * NC + lax.axis_index("c")
        row0 = wid * ROWS_W
        pltpu.sync_copy(idxa_hbm.at[pl.ds(row0, ROWS_W)], idxa_v)
        pltpu.sync_copy(idxb_hbm.at[pl.ds(row0, ROWS_W), pl.ds(0, GB)], idxb_v)

        def out_dst(r):
            return out_hbm.at[row0 + r, :, pl.ds(0, D)]

        def issue_gathers(r, p):
            pltpu.async_copy(
                table_hbm.at[idxa_v.at[r]],
                bufs[p].at[pl.ds(0, GA)],
                gsem[p],
            )
            pltpu.async_copy(
                table_hbm.at[idxb_v.at[r]],
                bufs[p].at[pl.ds(GA, GB)],
                gsem[p],
            )

        def wait_gathers(p):
            pltpu.make_async_copy(
                table_hbm.at[idxa_v.at[0]],
                bufs[p].at[pl.ds(0, GA)],
                gsem[p],
            ).wait()
            pltpu.make_async_copy(
                table_hbm.at[idxb_v.at[0]],
                bufs[p].at[pl.ds(GA, GB)],
                gsem[p],
            ).wait()

        def wait_write(p):
            pltpu.make_async_copy(bufs[p], out_dst(0), wsem[p]).wait()

        issue_gathers(0, 0)
        issue_gathers(1, 1)

        def body(m, carry):
            for j in range(NBUF):
                r = NBUF * m + j
                p = j
                p2 = (j + 2) % NBUF
                wait_gathers(p)
                pltpu.async_copy(bufs[p], out_dst(r), wsem[p])

                @pl.when(r >= 2)
                def _():
                    wait_write(p2)

                @pl.when(r + 2 < ROWS_W)
                def _():
                    issue_gathers(r + 2, p2)

            return carry

        lax.fori_loop(0, ROWS_W // NBUF, body, 0)
        wait_write(2)
        wait_write(3)

    return k


_sc_gather = _make_sc_gather()


def kernel(word_indices, word_embedding_weight):
    idx = word_indices.astype(jnp.int32)
    idx = jnp.where(idx < SPLIT, idx * 2, (idx - SPLIT) * 2 + 1)
    idxa = idx[:, :GA]
    idxb = jnp.pad(idx[:, GA:], ((0, 0), (0, GA - GB)))
    packed = _tc_linearize(word_embedding_weight.T)
    table_lin = packed.reshape(2 * SPLIT, D)
    out = _sc_gather(idxa, idxb, table_lin)
    return out[:, :, :D]
